# baseline (speedup 1.0000x reference)
import functools

import jax
import jax.numpy as jnp
from jax import lax
from jax.experimental import pallas as pl
from jax.experimental.pallas import tpu as pltpu
from jax.experimental.pallas import tpu_sc as plsc

MARGIN_SQ = 0.25

ROWS, COLS = 16384, 128
TOTAL = ROWS * COLS

SC_ROWS = 2560
TC_ROWS = ROWS - SC_ROWS
TC_TOTAL = TC_ROWS * COLS

NC, NS, L = 2, 16, 16
NW = NC * NS
PER_W = SC_ROWS * COLS // NW
CHUNK = PER_W // 2
NCHUNK = PER_W // CHUNK
UNROLL = 8
NACC = 4

TC_BLOCK = 3456
TC_GRID = TC_ROWS // TC_BLOCK


def _sc_partials(pred_flat, real_flat):
    mesh = plsc.VectorSubcoreMesh(core_axis_name="c", subcore_axis_name="s")

    @functools.partial(
        pl.kernel,
        mesh=mesh,
        out_type=[
            jax.ShapeDtypeStruct((NW, L), jnp.float32),
            jax.ShapeDtypeStruct((NW, L), jnp.float32),
        ],
        scratch_types=[
            pltpu.VMEM((2 * CHUNK,), jnp.float32),
            pltpu.VMEM((2 * CHUNK,), jnp.float32),
            pltpu.VMEM((L,), jnp.float32),
            pltpu.VMEM((L,), jnp.float32),
            pltpu.SemaphoreType.DMA,
            pltpu.SemaphoreType.DMA,
        ],
    )
    def body(pred_hbm, real_hbm, sq_out, cnt_out, pbuf, rbuf, sq_v, cnt_v,
             sem0, sem1):
        wid = lax.axis_index("s") * NC + lax.axis_index("c")
        base = TC_TOTAL + wid * PER_W
        sems = (sem0, sem1)

        def start(c):
            b = c % 2
            off = base + c * CHUNK
            hp = pltpu.async_copy(
                pred_hbm.at[pl.ds(off, CHUNK)],
                pbuf.at[pl.ds(b * CHUNK, CHUNK)], sems[b])
            hr = pltpu.async_copy(
                real_hbm.at[pl.ds(off, CHUNK)],
                rbuf.at[pl.ds(b * CHUNK, CHUNK)], sems[b])
            return hp, hr

        handles = [None] * NCHUNK
        handles[0] = start(0)

        zf = jnp.zeros((L,), jnp.float32)
        accs = (zf,) * (2 * NACC)

        for c in range(NCHUNK):
            if c + 1 < NCHUNK:
                handles[c + 1] = start(c + 1)
            hp, hr = handles[c]
            hp.wait()
            hr.wait()
            vbase = (c % 2) * CHUNK

            def vec_body(i, acc, vbase=vbase):
                sqs = list(acc[:NACC])
                cnts = list(acc[NACC:])
                o = vbase + i * (L * UNROLL)
                for u in range(UNROLL):
                    p = pbuf[pl.ds(o + u * L, L)]
                    r = rbuf[pl.ds(o + u * L, L)]
                    d = r - p
                    sq = d * d
                    m = sq > MARGIN_SQ
                    a = u % NACC
                    sqs[a] = sqs[a] + jnp.where(m, sq, 0.0)
                    cnts[a] = cnts[a] + jnp.where(m, 1.0, 0.0)
                return tuple(sqs) + tuple(cnts)

            accs = lax.fori_loop(0, CHUNK // (L * UNROLL), vec_body, accs)

        acc_sq = accs[0]
        acc_cnt = accs[NACC]
        for a in range(1, NACC):
            acc_sq = acc_sq + accs[a]
            acc_cnt = acc_cnt + accs[NACC + a]

        sq_v[...] = acc_sq
        cnt_v[...] = acc_cnt
        pltpu.sync_copy(sq_v, sq_out.at[wid])
        pltpu.sync_copy(cnt_v, cnt_out.at[wid])

    return body(pred_flat, real_flat)


def _tc_body(p_ref, r_ref, out_ref, acc_sq, acc_cnt):
    i = pl.program_id(0)
    p = p_ref[...]
    r = r_ref[...]
    d = r - p
    sq = d * d
    m = sq > MARGIN_SQ
    csq = jnp.sum(jnp.where(m, sq, 0.0).reshape(TC_BLOCK // 8, 8, COLS),
                  axis=0)
    ccnt = jnp.sum(jnp.where(m, 1.0, 0.0).reshape(TC_BLOCK // 8, 8, COLS),
                   axis=0)

    @pl.when(i == 0)
    def _():
        acc_sq[...] = csq
        acc_cnt[...] = ccnt

    @pl.when(i > 0)
    def _():
        acc_sq[...] += csq
        acc_cnt[...] += ccnt

    @pl.when(i == TC_GRID - 1)
    def _():
        out_ref[0, 0] = jnp.sum(acc_sq[...])
        out_ref[0, 1] = jnp.sum(acc_cnt[...])


def _tc_partials(pred, real):
    return pl.pallas_call(
        _tc_body,
        grid=(TC_GRID,),
        in_specs=[
            pl.BlockSpec((TC_BLOCK, COLS), lambda i: (i, 0)),
            pl.BlockSpec((TC_BLOCK, COLS), lambda i: (i, 0)),
        ],
        out_specs=pl.BlockSpec(memory_space=pltpu.SMEM),
        out_shape=jax.ShapeDtypeStruct((1, 2), jnp.float32),
        scratch_shapes=[
            pltpu.VMEM((8, COLS), jnp.float32),
            pltpu.VMEM((8, COLS), jnp.float32),
        ],
    )(pred, real)


def _combine_body(sc_sq_ref, sc_cnt_ref, tc_sn_ref, out_ref):
    s = jnp.sum(sc_sq_ref[...]) + tc_sn_ref[0, 0]
    n = jnp.sum(sc_cnt_ref[...]) + tc_sn_ref[0, 1]
    out_ref[0, 0] = jnp.where(n > 0.0, s / jnp.maximum(n, 1.0), 0.0)


def _combine(sc_sq, sc_cnt, tc_sn):
    return pl.pallas_call(
        _combine_body,
        in_specs=[
            pl.BlockSpec((NW, L), lambda: (0, 0)),
            pl.BlockSpec((NW, L), lambda: (0, 0)),
            pl.BlockSpec(memory_space=pltpu.SMEM),
        ],
        out_shape=jax.ShapeDtypeStruct((1, 1), jnp.float32),
        out_specs=pl.BlockSpec(memory_space=pltpu.SMEM),
    )(sc_sq, sc_cnt, tc_sn)


def kernel(pred, real):
    pred_flat = pred.reshape(TOTAL)
    real_flat = real.reshape(TOTAL)
    sc_sq, sc_cnt = _sc_partials(pred_flat, real_flat)
    tc_sn = _tc_partials(pred, real)
    out = _combine(sc_sq, sc_cnt, tc_sn)
    return out[0, 0]


# speedup vs baseline: 1.0143x; 1.0143x over previous
import functools

import jax
import jax.numpy as jnp
from jax import lax
from jax.experimental import pallas as pl
from jax.experimental.pallas import tpu as pltpu
from jax.experimental.pallas import tpu_sc as plsc

MARGIN_SQ = 0.25

ROWS, COLS = 16384, 128
TOTAL = ROWS * COLS

SC_ROWS = 2048
TC_ROWS = ROWS - SC_ROWS
TC_TOTAL = TC_ROWS * COLS

NC, NS, L = 2, 16, 16
NW = NC * NS
PER_W = SC_ROWS * COLS // NW
CHUNK = PER_W // 2
NCHUNK = PER_W // CHUNK
UNROLL = 8
NACC = 4

TC_BLOCK = 3584
TC_GRID = TC_ROWS // TC_BLOCK


def _sc_partials(pred_flat, real_flat):
    mesh = plsc.VectorSubcoreMesh(core_axis_name="c", subcore_axis_name="s")

    @functools.partial(
        pl.kernel,
        mesh=mesh,
        out_type=[
            jax.ShapeDtypeStruct((NW, L), jnp.float32),
            jax.ShapeDtypeStruct((NW, L), jnp.float32),
        ],
        scratch_types=[
            pltpu.VMEM((2 * CHUNK,), jnp.float32),
            pltpu.VMEM((2 * CHUNK,), jnp.float32),
            pltpu.VMEM((L,), jnp.float32),
            pltpu.VMEM((L,), jnp.float32),
            pltpu.SemaphoreType.DMA,
            pltpu.SemaphoreType.DMA,
        ],
    )
    def body(pred_hbm, real_hbm, sq_out, cnt_out, pbuf, rbuf, sq_v, cnt_v,
             sem0, sem1):
        wid = lax.axis_index("s") * NC + lax.axis_index("c")
        base = TC_TOTAL + wid * PER_W
        sems = (sem0, sem1)

        def start(c):
            b = c % 2
            off = base + c * CHUNK
            hp = pltpu.async_copy(
                pred_hbm.at[pl.ds(off, CHUNK)],
                pbuf.at[pl.ds(b * CHUNK, CHUNK)], sems[b])
            hr = pltpu.async_copy(
                real_hbm.at[pl.ds(off, CHUNK)],
                rbuf.at[pl.ds(b * CHUNK, CHUNK)], sems[b])
            return hp, hr

        handles = [None] * NCHUNK
        handles[0] = start(0)

        zf = jnp.zeros((L,), jnp.float32)
        accs = (zf,) * (2 * NACC)

        for c in range(NCHUNK):
            if c + 1 < NCHUNK:
                handles[c + 1] = start(c + 1)
            hp, hr = handles[c]
            hp.wait()
            hr.wait()
            vbase = (c % 2) * CHUNK

            def vec_body(i, acc, vbase=vbase):
                sqs = list(acc[:NACC])
                cnts = list(acc[NACC:])
                o = vbase + i * (L * UNROLL)
                for u in range(UNROLL):
                    p = pbuf[pl.ds(o + u * L, L)]
                    r = rbuf[pl.ds(o + u * L, L)]
                    d = r - p
                    sq = d * d
                    m = sq > MARGIN_SQ
                    a = u % NACC
                    sqs[a] = sqs[a] + jnp.where(m, sq, 0.0)
                    cnts[a] = cnts[a] + jnp.where(m, 1.0, 0.0)
                return tuple(sqs) + tuple(cnts)

            accs = lax.fori_loop(0, CHUNK // (L * UNROLL), vec_body, accs)

        acc_sq = accs[0]
        acc_cnt = accs[NACC]
        for a in range(1, NACC):
            acc_sq = acc_sq + accs[a]
            acc_cnt = acc_cnt + accs[NACC + a]

        sq_v[...] = acc_sq
        cnt_v[...] = acc_cnt
        pltpu.sync_copy(sq_v, sq_out.at[wid])
        pltpu.sync_copy(cnt_v, cnt_out.at[wid])

    return body(pred_flat, real_flat)


def _tc_body(p_ref, r_ref, out_ref, acc_sq, acc_cnt):
    i = pl.program_id(0)
    p = p_ref[...]
    r = r_ref[...]
    d = r - p
    sq = d * d
    m = sq > MARGIN_SQ
    csq = jnp.sum(jnp.where(m, sq, 0.0).reshape(TC_BLOCK // 8, 8, COLS),
                  axis=0)
    ccnt = jnp.sum(jnp.where(m, 1.0, 0.0).reshape(TC_BLOCK // 8, 8, COLS),
                   axis=0)

    @pl.when(i == 0)
    def _():
        acc_sq[...] = csq
        acc_cnt[...] = ccnt

    @pl.when(i > 0)
    def _():
        acc_sq[...] += csq
        acc_cnt[...] += ccnt

    @pl.when(i == TC_GRID - 1)
    def _():
        out_ref[0, 0] = jnp.sum(acc_sq[...])
        out_ref[0, 1] = jnp.sum(acc_cnt[...])


def _tc_partials(pred, real):
    return pl.pallas_call(
        _tc_body,
        grid=(TC_GRID,),
        in_specs=[
            pl.BlockSpec((TC_BLOCK, COLS), lambda i: (i, 0)),
            pl.BlockSpec((TC_BLOCK, COLS), lambda i: (i, 0)),
        ],
        out_specs=pl.BlockSpec(memory_space=pltpu.SMEM),
        out_shape=jax.ShapeDtypeStruct((1, 2), jnp.float32),
        scratch_shapes=[
            pltpu.VMEM((8, COLS), jnp.float32),
            pltpu.VMEM((8, COLS), jnp.float32),
        ],
    )(pred, real)


def _combine_body(sc_sq_ref, sc_cnt_ref, tc_sn_ref, out_ref):
    s = jnp.sum(sc_sq_ref[...]) + tc_sn_ref[0, 0]
    n = jnp.sum(sc_cnt_ref[...]) + tc_sn_ref[0, 1]
    out_ref[0, 0] = jnp.where(n > 0.0, s / jnp.maximum(n, 1.0), 0.0)


def _combine(sc_sq, sc_cnt, tc_sn):
    return pl.pallas_call(
        _combine_body,
        in_specs=[
            pl.BlockSpec((NW, L), lambda: (0, 0)),
            pl.BlockSpec((NW, L), lambda: (0, 0)),
            pl.BlockSpec(memory_space=pltpu.SMEM),
        ],
        out_shape=jax.ShapeDtypeStruct((1, 1), jnp.float32),
        out_specs=pl.BlockSpec(memory_space=pltpu.SMEM),
    )(sc_sq, sc_cnt, tc_sn)


def kernel(pred, real):
    pred_flat = pred.reshape(TOTAL)
    real_flat = real.reshape(TOTAL)
    sc_sq, sc_cnt = _sc_partials(pred_flat, real_flat)
    tc_sn = _tc_partials(pred, real)
    out = _combine(sc_sq, sc_cnt, tc_sn)
    return out[0, 0]
